# baseline (speedup 1.0000x reference)
import jax
import jax.numpy as jnp
from jax import lax
from jax.experimental import pallas as pl


ROWS = 4096
COLS = 20
VOCAB = 1000
BLOCK_R = 64


def _onehot_block(x_ref, out_ref):
    xv = x_ref[...]
    iota = lax.broadcasted_iota(jnp.int32, (BLOCK_R, COLS, VOCAB), 2)
    out_ref[...] = (xv == iota).astype(jnp.int32)


def kernel(x):
    x3 = x.astype(jnp.int32)[:, :, None]
    out = pl.pallas_call(
        _onehot_block,
        grid=(ROWS // BLOCK_R,),
        in_specs=[pl.BlockSpec((BLOCK_R, COLS, 1), lambda i: (i, 0, 0))],
        out_specs=pl.BlockSpec((BLOCK_R, COLS, VOCAB), lambda i: (i, 0, 0)),
        out_shape=jax.ShapeDtypeStruct((ROWS, COLS, VOCAB), jnp.int32),
    )(x3)
    return out


# speedup vs baseline: 1.1076x; 1.1076x over previous
import jax
import jax.numpy as jnp
from jax import lax
from jax.experimental import pallas as pl


ROWS = 4096
COLS = 20
VOCAB = 1000
BLOCK_R = 128


def _onehot_block(x_ref, out_ref):
    xv = x_ref[...][:, :, None]
    iota = lax.broadcasted_iota(jnp.int32, (BLOCK_R, COLS, VOCAB), 2)
    out_ref[...] = (xv == iota).astype(jnp.int32)


def kernel(x):
    x32 = x.astype(jnp.int32)
    out = pl.pallas_call(
        _onehot_block,
        grid=(ROWS // BLOCK_R,),
        in_specs=[pl.BlockSpec((BLOCK_R, COLS), lambda i: (i, 0))],
        out_specs=pl.BlockSpec((BLOCK_R, COLS, VOCAB), lambda i: (i, 0, 0)),
        out_shape=jax.ShapeDtypeStruct((ROWS, COLS, VOCAB), jnp.int32),
    )(x32)
    return out


# speedup vs baseline: 1.1104x; 1.0025x over previous
import jax
import jax.numpy as jnp
from jax import lax
from jax.experimental import pallas as pl
from jax.experimental.pallas import tpu as pltpu


ROWS = 4096
COLS = 20
VOCAB = 1000
BLOCK = 128
NBUF = 2
K = 4
CH = BLOCK // K
STEP_ROWS = BLOCK * NBUF


def _dma(scratch, out_hbm, sems, h, k, row0):
    return pltpu.make_async_copy(
        scratch.at[h, pl.ds(k * CH, CH)],
        out_hbm.at[pl.ds(row0 + k * CH, CH)],
        sems.at[h, k],
    )


def _body(x_ref, out_hbm, scratch, sems):
    i = pl.program_id(0)
    ng = pl.num_programs(0)
    iota = lax.broadcasted_iota(jnp.int32, (BLOCK, COLS, VOCAB), 2)

    for h in range(NBUF):
        row0 = i * STEP_ROWS + h * BLOCK

        @pl.when(i >= 1)
        def _wait_prev(h=h, row0=row0):
            for k in range(K):
                _dma(scratch, out_hbm, sems, h, k, row0 - STEP_ROWS).wait()

        xv = x_ref[pl.ds(h * BLOCK, BLOCK), :][:, :, None]
        scratch[h] = (xv == iota).astype(jnp.int32)
        for k in range(K):
            _dma(scratch, out_hbm, sems, h, k, row0).start()

    @pl.when(i == ng - 1)
    def _drain():
        for h in range(NBUF):
            row0 = i * STEP_ROWS + h * BLOCK
            for k in range(K):
                _dma(scratch, out_hbm, sems, h, k, row0).wait()


def kernel(x):
    x32 = x.astype(jnp.int32)
    out = pl.pallas_call(
        _body,
        grid=(ROWS // STEP_ROWS,),
        in_specs=[pl.BlockSpec((STEP_ROWS, COLS), lambda i: (i, 0))],
        out_specs=pl.BlockSpec(memory_space=pl.ANY),
        out_shape=jax.ShapeDtypeStruct((ROWS, COLS, VOCAB), jnp.int32),
        scratch_shapes=[
            pltpu.VMEM((NBUF, BLOCK, COLS, VOCAB), jnp.int32),
            pltpu.SemaphoreType.DMA((NBUF, K)),
        ],
    )(x32)
    return out


# speedup vs baseline: 4.8095x; 4.3314x over previous
import jax
import jax.numpy as jnp
from jax import lax
from jax.experimental import pallas as pl


ROWS = 4096
COLS = 20
VOCAB = 1000


def _onehot_block(x_ref, out_ref):
    xv = x_ref[...]
    iota = lax.broadcasted_iota(jnp.int32, (1, VOCAB, ROWS), 1)
    out_ref[...] = (xv == iota).astype(jnp.int32)


def kernel(x):
    xt = x.astype(jnp.int32).T[:, None, :]
    out_t = pl.pallas_call(
        _onehot_block,
        grid=(COLS,),
        in_specs=[pl.BlockSpec((1, 1, ROWS), lambda c: (c, 0, 0))],
        out_specs=pl.BlockSpec((1, VOCAB, ROWS), lambda c: (c, 0, 0)),
        out_shape=jax.ShapeDtypeStruct((COLS, VOCAB, ROWS), jnp.int32),
    )(xt)
    return jnp.transpose(out_t, (2, 0, 1))


# speedup vs baseline: 4.8792x; 1.0145x over previous
import jax
import jax.numpy as jnp
from jax import lax
from jax.experimental import pallas as pl


ROWS = 4096
COLS = 20
VOCAB = 1000


def _onehot_block(x_ref, out_ref):
    c = pl.program_id(0)
    xv = x_ref[pl.ds(c, 1), :][:, None, :]
    iota = lax.broadcasted_iota(jnp.int32, (1, VOCAB, ROWS), 1)
    out_ref[...] = (xv == iota).astype(jnp.int32)


def kernel(x):
    xt = x.astype(jnp.int32).T
    out_t = pl.pallas_call(
        _onehot_block,
        grid=(COLS,),
        in_specs=[pl.BlockSpec((COLS, ROWS), lambda c: (0, 0))],
        out_specs=pl.BlockSpec((1, VOCAB, ROWS), lambda c: (c, 0, 0)),
        out_shape=jax.ShapeDtypeStruct((COLS, VOCAB, ROWS), jnp.int32),
    )(xt)
    return jnp.transpose(out_t, (2, 0, 1))
